# baseline (speedup 1.0000x reference)
import functools

import jax
import jax.numpy as jnp
from jax import lax
from jax.experimental import pallas as pl
from jax.experimental.pallas import tpu as pltpu
from jax.experimental.pallas import tpu_sc as plsc

N = 10000
E = 320000
NW = 32
CHUNK = 128
C = 80
E_PAD = NW * C * CHUNK
N_ACC = 10112
ROWS_PER_TILE = N_ACC // 16
N8 = N + 8


def _make_sc_agg(dw):
  mesh = plsc.VectorSubcoreMesh(core_axis_name="c", subcore_axis_name="s")

  def body(y_hbm, src_hbm, dst_hbm, z_hbm, out_hbm, src_v, dst_v, rows_v,
           acc, gsem):
    c = lax.axis_index("c")
    s = lax.axis_index("s")
    wid = s * 2 + c
    r0 = s * ROWS_PER_TILE
    pltpu.sync_copy(z_hbm.at[pl.ds(r0, ROWS_PER_TILE)],
                    acc.at[pl.ds(r0, ROWS_PER_TILE)])
    pltpu.sync_copy(src_hbm.at[wid], src_v)
    pltpu.sync_copy(dst_hbm.at[wid], dst_v)
    plsc.subcore_barrier()

    def fire_g(j, b):
      pltpu.async_copy(y_hbm.at[src_v.at[j]], rows_v.at[b], gsem)

    fire_g(0, 0)

    def step(j, carry):
      b = j % 2
      pltpu.make_async_copy(y_hbm.at[src_v.at[j]], rows_v.at[b], gsem).wait()
      lax.cond(j < C - 1, lambda: fire_g(j + 1, 1 - b), lambda: None)
      pltpu.sync_copy(rows_v.at[b], acc.at[dst_v.at[j]], add=True)
      return carry

    lax.fori_loop(0, C, step, 0)

    plsc.subcore_barrier()
    pltpu.sync_copy(acc.at[pl.ds(r0, ROWS_PER_TILE)],
                    out_hbm.at[c, pl.ds(r0, ROWS_PER_TILE)])

  return pl.kernel(
      body,
      out_type=jax.ShapeDtypeStruct((2, N_ACC, dw), jnp.float32),
      mesh=mesh,
      compiler_params=pltpu.CompilerParams(use_tc_tiling_on_sc=False),
      scratch_types=[
          pltpu.VMEM((C, CHUNK), jnp.int32),
          pltpu.VMEM((C, CHUNK), jnp.int32),
          pltpu.VMEM((2, CHUNK, dw), jnp.float32),
          pltpu.VMEM_SHARED((N_ACC, dw), jnp.float32),
          pltpu.SemaphoreType.DMA,
      ],
  )


_sc_agg_80 = _make_sc_agg(80)
_sc_agg_32 = _make_sc_agg(32)
_sc_agg_16 = _make_sc_agg(16)


def _tc0_body(x_ref, w_ref, o_ref):
  y = jnp.dot(x_ref[...], w_ref[...], preferred_element_type=jnp.float32)
  yp = jnp.concatenate(
      [y, jnp.ones((N, 1), jnp.float32), jnp.zeros((N, 15), jnp.float32)],
      axis=1)
  o_ref[...] = jnp.concatenate([yp, jnp.zeros((8, 80), jnp.float32)], axis=0)


_tc0 = pl.pallas_call(
    _tc0_body, out_shape=jax.ShapeDtypeStruct((N8, 80), jnp.float32))


def _tc1_body(agg_ref, x_ref, wr_ref, b_ref, wl2_ref, h_ref, y2_ref, cnt_ref):
  a = agg_ref[0] + agg_ref[1]
  cnt = jnp.maximum(a[:, 64:65], 1.0)
  mean = a[:, :64] / cnt
  h = jnp.maximum(
      mean + jnp.dot(x_ref[...], wr_ref[...],
                     preferred_element_type=jnp.float32) + b_ref[...], 0.0)
  h_ref[...] = h
  y2 = jnp.dot(h, wl2_ref[...], preferred_element_type=jnp.float32)
  y2_ref[...] = jnp.concatenate([y2, jnp.zeros((8, 32), jnp.float32)], axis=0)
  cnt_ref[...] = cnt


_tc1 = pl.pallas_call(
    _tc1_body,
    out_shape=(
        jax.ShapeDtypeStruct((N, 64), jnp.float32),
        jax.ShapeDtypeStruct((N8, 32), jnp.float32),
        jax.ShapeDtypeStruct((N, 1), jnp.float32),
    ))


def _tc2_body(agg_ref, h1_ref, cnt_ref, wr_ref, b_ref, wl3_ref, h_ref, y3_ref):
  a = agg_ref[0] + agg_ref[1]
  mean = a / cnt_ref[...]
  h = jnp.maximum(
      mean + jnp.dot(h1_ref[...], wr_ref[...],
                     preferred_element_type=jnp.float32) + b_ref[...], 0.0)
  h_ref[...] = h
  y3 = jnp.dot(h, wl3_ref[...], preferred_element_type=jnp.float32)
  y3_ref[...] = jnp.concatenate([y3, jnp.zeros((8, 16), jnp.float32)], axis=0)


_tc2 = pl.pallas_call(
    _tc2_body,
    out_shape=(
        jax.ShapeDtypeStruct((N, 32), jnp.float32),
        jax.ShapeDtypeStruct((N8, 16), jnp.float32),
    ))


def _tc3_body(agg_ref, h2_ref, cnt_ref, wr_ref, b_ref, wh_ref, bh_ref, o_ref):
  a = agg_ref[0] + agg_ref[1]
  mean = a / cnt_ref[...]
  h = jnp.maximum(
      mean + jnp.dot(h2_ref[...], wr_ref[...],
                     preferred_element_type=jnp.float32) + b_ref[...], 0.0)
  o_ref[...] = jnp.dot(
      h, wh_ref[...], preferred_element_type=jnp.float32) + bh_ref[...]


_tc3 = pl.pallas_call(
    _tc3_body, out_shape=jax.ShapeDtypeStruct((N, 2), jnp.float32))


@jax.jit
def _run(x, edge_index, W1l, W1r, b1, W2l, W2r, b2, W3l, W3r, b3, Wreg, breg,
         Wcls, bcls):
  ei = edge_index.astype(jnp.int32)
  pad = E_PAD - E
  src = jnp.concatenate([ei[0], jnp.full((pad,), N, jnp.int32)])
  dst = jnp.concatenate([ei[1], jnp.zeros((pad,), jnp.int32)])
  src = src.reshape(NW, C, CHUNK)
  dst = dst.reshape(NW, C, CHUNK)
  z80 = jnp.zeros((N_ACC, 80), jnp.float32)
  z32 = jnp.zeros((N_ACC, 32), jnp.float32)
  z16 = jnp.zeros((N_ACC, 16), jnp.float32)

  y1p = _tc0(x, W1l)
  agg1 = _sc_agg_80(y1p, src, dst, z80)[:, :N]
  h1, y2p, cnt = _tc1(agg1, x, W1r, b1.reshape(1, 64), W2l)
  agg2 = _sc_agg_32(y2p, src, dst, z32)[:, :N]
  h2, y3p = _tc2(agg2, h1, cnt, W2r, b2.reshape(1, 32), W3l)
  agg3 = _sc_agg_16(y3p, src, dst, z16)[:, :N]
  wh = jnp.concatenate([Wreg, Wcls], axis=1)
  bh = jnp.stack([breg[0], bcls[0]]).reshape(1, 2)
  out = _tc3(agg3, h2, cnt, W3r, b3.reshape(1, 16), wh, bh)
  return out[:, 0], out[:, 1]


def kernel(x, edge_index, W1l, W1r, b1, W2l, W2r, b2, W3l, W3r, b3, Wreg,
           breg, Wcls, bcls):
  return _run(x, edge_index, W1l, W1r, b1, W2l, W2r, b2, W3l, W3r, b3, Wreg,
              breg, Wcls, bcls)


# speedup vs baseline: 1.2532x; 1.2532x over previous
import functools

import jax
import jax.numpy as jnp
from jax import lax
from jax.experimental import pallas as pl
from jax.experimental.pallas import tpu as pltpu
from jax.experimental.pallas import tpu_sc as plsc

N = 10000
E = 320000
NW = 32
CHUNK = 128
C = 80
E_PAD = NW * C * CHUNK
N_ACC = 10112
ROWS_PER_TILE = N_ACC // 16
N8 = N + 16
Y_ROWS_PER_TILE = N8 // 16


def _make_sc_agg(dw, stage_y):
  mesh = plsc.VectorSubcoreMesh(core_axis_name="c", subcore_axis_name="s")

  def body(y_hbm, src_hbm, dst_hbm, z_hbm, out_hbm, src_v, dst_v, rows_v,
           acc, y_spm, gsem):
    c = lax.axis_index("c")
    s = lax.axis_index("s")
    wid = s * 2 + c
    r0 = s * ROWS_PER_TILE
    pltpu.sync_copy(z_hbm.at[pl.ds(r0, ROWS_PER_TILE)],
                    acc.at[pl.ds(r0, ROWS_PER_TILE)])
    if stage_y:
      y0 = s * Y_ROWS_PER_TILE
      pltpu.sync_copy(y_hbm.at[pl.ds(y0, Y_ROWS_PER_TILE)],
                      y_spm.at[pl.ds(y0, Y_ROWS_PER_TILE)])
      y_src = y_spm
    else:
      y_src = y_hbm
    pltpu.sync_copy(src_hbm.at[wid], src_v)
    pltpu.sync_copy(dst_hbm.at[wid], dst_v)
    plsc.subcore_barrier()

    def step(j, carry):
      pltpu.async_copy(y_src.at[src_v.at[j]], rows_v, gsem).wait()
      pltpu.sync_copy(rows_v, acc.at[dst_v.at[j]], add=True)
      return carry

    lax.fori_loop(0, C, step, 0)

    plsc.subcore_barrier()
    pltpu.sync_copy(acc.at[pl.ds(r0, ROWS_PER_TILE)],
                    out_hbm.at[c, pl.ds(r0, ROWS_PER_TILE)])

  scratch = [
      pltpu.VMEM((C, CHUNK), jnp.int32),
      pltpu.VMEM((C, CHUNK), jnp.int32),
      pltpu.VMEM((CHUNK, dw), jnp.float32),
      pltpu.VMEM_SHARED((N_ACC, dw), jnp.float32),
      pltpu.VMEM_SHARED((N8, dw) if stage_y else (8, dw), jnp.float32),
      pltpu.SemaphoreType.DMA,
  ]
  return pl.kernel(
      body,
      out_type=jax.ShapeDtypeStruct((2, N_ACC, dw), jnp.float32),
      mesh=mesh,
      compiler_params=pltpu.CompilerParams(use_tc_tiling_on_sc=False),
      scratch_types=scratch,
  )


_sc_agg_80 = _make_sc_agg(80, stage_y=False)
_sc_agg_32 = _make_sc_agg(32, stage_y=True)
_sc_agg_16 = _make_sc_agg(16, stage_y=True)


def _tc0_body(x_ref, w_ref, o_ref):
  y = jnp.dot(x_ref[...], w_ref[...], preferred_element_type=jnp.float32)
  yp = jnp.concatenate(
      [y, jnp.ones((N, 1), jnp.float32), jnp.zeros((N, 15), jnp.float32)],
      axis=1)
  o_ref[...] = jnp.concatenate([yp, jnp.zeros((16, 80), jnp.float32)], axis=0)


_tc0 = pl.pallas_call(
    _tc0_body, out_shape=jax.ShapeDtypeStruct((N8, 80), jnp.float32))


def _tc1_body(agg_ref, x_ref, wr_ref, b_ref, wl2_ref, h_ref, y2_ref, cnt_ref):
  a = agg_ref[0] + agg_ref[1]
  cnt = jnp.maximum(a[:, 64:65], 1.0)
  mean = a[:, :64] / cnt
  h = jnp.maximum(
      mean + jnp.dot(x_ref[...], wr_ref[...],
                     preferred_element_type=jnp.float32) + b_ref[...], 0.0)
  h_ref[...] = h
  y2 = jnp.dot(h, wl2_ref[...], preferred_element_type=jnp.float32)
  y2_ref[...] = jnp.concatenate([y2, jnp.zeros((16, 32), jnp.float32)], axis=0)
  cnt_ref[...] = cnt


_tc1 = pl.pallas_call(
    _tc1_body,
    out_shape=(
        jax.ShapeDtypeStruct((N, 64), jnp.float32),
        jax.ShapeDtypeStruct((N8, 32), jnp.float32),
        jax.ShapeDtypeStruct((N, 1), jnp.float32),
    ))


def _tc2_body(agg_ref, h1_ref, cnt_ref, wr_ref, b_ref, wl3_ref, h_ref, y3_ref):
  a = agg_ref[0] + agg_ref[1]
  mean = a / cnt_ref[...]
  h = jnp.maximum(
      mean + jnp.dot(h1_ref[...], wr_ref[...],
                     preferred_element_type=jnp.float32) + b_ref[...], 0.0)
  h_ref[...] = h
  y3 = jnp.dot(h, wl3_ref[...], preferred_element_type=jnp.float32)
  y3_ref[...] = jnp.concatenate([y3, jnp.zeros((16, 16), jnp.float32)], axis=0)


_tc2 = pl.pallas_call(
    _tc2_body,
    out_shape=(
        jax.ShapeDtypeStruct((N, 32), jnp.float32),
        jax.ShapeDtypeStruct((N8, 16), jnp.float32),
    ))


def _tc3_body(agg_ref, h2_ref, cnt_ref, wr_ref, b_ref, wh_ref, bh_ref, o_ref):
  a = agg_ref[0] + agg_ref[1]
  mean = a / cnt_ref[...]
  h = jnp.maximum(
      mean + jnp.dot(h2_ref[...], wr_ref[...],
                     preferred_element_type=jnp.float32) + b_ref[...], 0.0)
  o_ref[...] = jnp.dot(
      h, wh_ref[...], preferred_element_type=jnp.float32) + bh_ref[...]


_tc3 = pl.pallas_call(
    _tc3_body, out_shape=jax.ShapeDtypeStruct((N, 2), jnp.float32))


@jax.jit
def _run(x, edge_index, W1l, W1r, b1, W2l, W2r, b2, W3l, W3r, b3, Wreg, breg,
         Wcls, bcls):
  ei = edge_index.astype(jnp.int32)
  pad = E_PAD - E
  src = jnp.concatenate([ei[0], jnp.full((pad,), N, jnp.int32)])
  dst = jnp.concatenate([ei[1], jnp.zeros((pad,), jnp.int32)])
  src = src.reshape(NW, C, CHUNK)
  dst = dst.reshape(NW, C, CHUNK)
  z80 = jnp.zeros((N_ACC, 80), jnp.float32)
  z32 = jnp.zeros((N_ACC, 32), jnp.float32)
  z16 = jnp.zeros((N_ACC, 16), jnp.float32)

  y1p = _tc0(x, W1l)
  agg1 = _sc_agg_80(y1p, src, dst, z80)[:, :N]
  h1, y2p, cnt = _tc1(agg1, x, W1r, b1.reshape(1, 64), W2l)
  agg2 = _sc_agg_32(y2p, src, dst, z32)[:, :N]
  h2, y3p = _tc2(agg2, h1, cnt, W2r, b2.reshape(1, 32), W3l)
  agg3 = _sc_agg_16(y3p, src, dst, z16)[:, :N]
  wh = jnp.concatenate([Wreg, Wcls], axis=1)
  bh = jnp.stack([breg[0], bcls[0]]).reshape(1, 2)
  out = _tc3(agg3, h2, cnt, W3r, b3.reshape(1, 16), wh, bh)
  return out[:, 0], out[:, 1]


def kernel(x, edge_index, W1l, W1r, b1, W2l, W2r, b2, W3l, W3r, b3, Wreg,
           breg, Wcls, bcls):
  return _run(x, edge_index, W1l, W1r, b1, W2l, W2r, b2, W3l, W3r, b3, Wreg,
              breg, Wcls, bcls)


# speedup vs baseline: 2.4848x; 1.9828x over previous
import jax
import jax.numpy as jnp
from jax import lax
from jax.experimental import pallas as pl
from jax.experimental.pallas import tpu as pltpu
from jax.experimental.pallas import tpu_sc as plsc

N = 10000
E = 320000
NW = 32
CHUNK = 256
EC = E // CHUNK
WCH = EC // NW
C = WCH + 1
N_ACC = 10112
ROWS_PER_TILE = N_ACC // 16
N8 = N + 16
Y_ROWS_PER_TILE = N8 // 16


def _make_sc_agg(dw, stage_y):
  mesh = plsc.VectorSubcoreMesh(core_axis_name="c", subcore_axis_name="s")

  def body(y_hbm, edges_hbm, z_hbm, out_hbm, src_v, dst_v, rows_v,
           acc, y_spm, gsem):
    c = lax.axis_index("c")
    s = lax.axis_index("s")
    wid = s * 2 + c
    r0 = s * ROWS_PER_TILE
    pltpu.sync_copy(z_hbm.at[pl.ds(r0, ROWS_PER_TILE)],
                    acc.at[pl.ds(r0, ROWS_PER_TILE)])
    if stage_y:
      y0 = s * Y_ROWS_PER_TILE
      pltpu.sync_copy(y_hbm.at[pl.ds(y0, Y_ROWS_PER_TILE)],
                      y_spm.at[pl.ds(y0, Y_ROWS_PER_TILE)])
      y_src = y_spm
    else:
      y_src = y_hbm
    pltpu.sync_copy(edges_hbm.at[0, pl.ds(wid * WCH, WCH)],
                    src_v.at[pl.ds(0, WCH)])
    pltpu.sync_copy(edges_hbm.at[1, pl.ds(wid * WCH, WCH)],
                    dst_v.at[pl.ds(0, WCH)])
    lax.cond(
        wid < EC - NW * WCH,
        lambda: (pltpu.sync_copy(edges_hbm.at[0, NW * WCH + wid],
                                 src_v.at[WCH]),
                 pltpu.sync_copy(edges_hbm.at[1, NW * WCH + wid],
                                 dst_v.at[WCH]))[0],
        lambda: None)
    nch = WCH + jnp.where(wid < EC - NW * WCH, 1, 0)
    plsc.subcore_barrier()

    def step(j, carry):
      pltpu.async_copy(y_src.at[src_v.at[j]], rows_v, gsem).wait()
      pltpu.sync_copy(rows_v, acc.at[dst_v.at[j]], add=True)
      return carry

    lax.fori_loop(0, nch, step, 0)

    plsc.subcore_barrier()
    pltpu.sync_copy(acc.at[pl.ds(r0, ROWS_PER_TILE)],
                    out_hbm.at[c, pl.ds(r0, ROWS_PER_TILE)])

  scratch = [
      pltpu.VMEM((C, CHUNK), jnp.int32),
      pltpu.VMEM((C, CHUNK), jnp.int32),
      pltpu.VMEM((CHUNK, dw), jnp.float32),
      pltpu.VMEM_SHARED((N_ACC, dw), jnp.float32),
      pltpu.VMEM_SHARED((N8, dw) if stage_y else (8, dw), jnp.float32),
      pltpu.SemaphoreType.DMA,
  ]
  return pl.kernel(
      body,
      out_type=jax.ShapeDtypeStruct((2, N_ACC, dw), jnp.float32),
      mesh=mesh,
      compiler_params=pltpu.CompilerParams(use_tc_tiling_on_sc=False),
      scratch_types=scratch,
  )


_sc_agg_32 = _make_sc_agg(32, stage_y=True)
_sc_agg_16 = _make_sc_agg(16, stage_y=True)


def _make_sc_agg1():
  mesh = plsc.VectorSubcoreMesh(core_axis_name="c", subcore_axis_name="s")

  def body(y_hbm, edges_hbm, z_hbm, agg_out, cnt_out, src_v, dst_v,
           rows_v, cnt_v, acc, y_spm, gsem):
    c = lax.axis_index("c")
    s = lax.axis_index("s")
    wid = s * 2 + c
    r0 = s * ROWS_PER_TILE
    pltpu.sync_copy(z_hbm.at[pl.ds(r0, ROWS_PER_TILE)],
                    acc.at[pl.ds(r0, ROWS_PER_TILE)])
    y0 = s * Y_ROWS_PER_TILE
    pltpu.sync_copy(y_hbm.at[pl.ds(y0, Y_ROWS_PER_TILE)],
                    y_spm.at[pl.ds(y0, Y_ROWS_PER_TILE)])
    pltpu.sync_copy(edges_hbm.at[0, pl.ds(wid * WCH, WCH)],
                    src_v.at[pl.ds(0, WCH)])
    pltpu.sync_copy(edges_hbm.at[1, pl.ds(wid * WCH, WCH)],
                    dst_v.at[pl.ds(0, WCH)])
    lax.cond(
        wid < EC - NW * WCH,
        lambda: (pltpu.sync_copy(edges_hbm.at[0, NW * WCH + wid],
                                 src_v.at[WCH]),
                 pltpu.sync_copy(edges_hbm.at[1, NW * WCH + wid],
                                 dst_v.at[WCH]))[0],
        lambda: None)
    nch = WCH + jnp.where(wid < EC - NW * WCH, 1, 0)

    def zero_cnt(i, carry):
      cnt_v[pl.ds(16 * i, 16)] = jnp.zeros((16,), jnp.float32)
      return carry

    lax.fori_loop(0, N8 // 16, zero_cnt, 0)
    plsc.subcore_barrier()

    ones16 = jnp.full((16,), 1.0, jnp.float32)

    def step(j, carry):
      cp = pltpu.async_copy(y_spm.at[src_v.at[j]], rows_v, gsem)
      drow = dst_v.at[j]
      for k in range(CHUNK // 16):
        idx = drow[pl.ds(16 * k, 16)]
        plsc.addupdate_scatter(cnt_v, [idx], ones16)
      cp.wait()
      pltpu.sync_copy(rows_v, acc.at[dst_v.at[j]], add=True)
      return carry

    lax.fori_loop(0, nch, step, 0)
    pltpu.sync_copy(cnt_v, cnt_out.at[wid])
    plsc.subcore_barrier()
    pltpu.sync_copy(acc.at[pl.ds(r0, ROWS_PER_TILE)],
                    agg_out.at[c, pl.ds(r0, ROWS_PER_TILE)])

  return pl.kernel(
      body,
      out_type=(jax.ShapeDtypeStruct((2, N_ACC, 64), jnp.float32),
                jax.ShapeDtypeStruct((NW, N8), jnp.float32)),
      mesh=mesh,
      compiler_params=pltpu.CompilerParams(use_tc_tiling_on_sc=False,
                                           needs_layout_passes=False),
      scratch_types=[
          pltpu.VMEM((C, CHUNK), jnp.int32),
          pltpu.VMEM((C, CHUNK), jnp.int32),
          pltpu.VMEM((CHUNK, 64), jnp.float32),
          pltpu.VMEM((N8,), jnp.float32),
          pltpu.VMEM_SHARED((N_ACC, 64), jnp.float32),
          pltpu.VMEM_SHARED((N8, 64), jnp.float32),
          pltpu.SemaphoreType.DMA,
      ],
  )


_sc_agg1 = _make_sc_agg1()


def _tc0_body(x_ref, w_ref, o_ref):
  y = jnp.dot(x_ref[...], w_ref[...], preferred_element_type=jnp.float32)
  o_ref[...] = jnp.concatenate([y, jnp.zeros((16, 64), jnp.float32)], axis=0)


_tc0 = pl.pallas_call(
    _tc0_body, out_shape=jax.ShapeDtypeStruct((N8, 64), jnp.float32))


def _tcr_body(h_ref, w_ref, b_ref, r_ref):
  r_ref[...] = jnp.dot(h_ref[...], w_ref[...],
                       preferred_element_type=jnp.float32) + b_ref[...]


_tcr = pl.pallas_call(
    _tcr_body, out_shape=jax.ShapeDtypeStruct((N, 64), jnp.float32))


def _tc1_body(agg_ref, cnts_ref, r1_ref, wl2_ref, wr2_ref, b2_ref,
              y2_ref, r2_ref, cnt_ref):
  a = agg_ref[0, :N] + agg_ref[1, :N]
  ccol = lax.dot_general(cnts_ref[...], jnp.ones((NW, 1), jnp.float32),
                         dimension_numbers=(((0,), (0,)), ((), ())))
  cnt = jnp.maximum(ccol[:N], 1.0)
  h = jnp.maximum(a / cnt + r1_ref[...], 0.0)
  y2 = jnp.dot(h, wl2_ref[...], preferred_element_type=jnp.float32)
  y2_ref[...] = jnp.concatenate([y2, jnp.zeros((16, 32), jnp.float32)], axis=0)
  r2_ref[...] = jnp.dot(h, wr2_ref[...],
                        preferred_element_type=jnp.float32) + b2_ref[...]
  cnt_ref[...] = cnt


_tc1 = pl.pallas_call(
    _tc1_body,
    out_shape=(
        jax.ShapeDtypeStruct((N8, 32), jnp.float32),
        jax.ShapeDtypeStruct((N, 32), jnp.float32),
        jax.ShapeDtypeStruct((N, 1), jnp.float32),
    ))


def _tc2_body(agg_ref, r2_ref, cnt_ref, wl3_ref, wr3_ref, b3_ref,
              y3_ref, r3_ref):
  a = agg_ref[0, :N] + agg_ref[1, :N]
  h = jnp.maximum(a / cnt_ref[...] + r2_ref[...], 0.0)
  y3 = jnp.dot(h, wl3_ref[...], preferred_element_type=jnp.float32)
  y3_ref[...] = jnp.concatenate([y3, jnp.zeros((16, 16), jnp.float32)], axis=0)
  r3_ref[...] = jnp.dot(h, wr3_ref[...],
                        preferred_element_type=jnp.float32) + b3_ref[...]


_tc2 = pl.pallas_call(
    _tc2_body,
    out_shape=(
        jax.ShapeDtypeStruct((N8, 16), jnp.float32),
        jax.ShapeDtypeStruct((N, 16), jnp.float32),
    ))


def _tc3_body(agg_ref, r3_ref, cnt_ref, wh_ref, bh_ref, o_ref):
  a = agg_ref[0, :N] + agg_ref[1, :N]
  h = jnp.maximum(a / cnt_ref[...] + r3_ref[...], 0.0)
  o_ref[...] = jnp.dot(
      h, wh_ref[...], preferred_element_type=jnp.float32) + bh_ref[...]


_tc3 = pl.pallas_call(
    _tc3_body, out_shape=jax.ShapeDtypeStruct((N, 2), jnp.float32))


@jax.jit
def _run(x, edge_index, W1l, W1r, b1, W2l, W2r, b2, W3l, W3r, b3, Wreg, breg,
         Wcls, bcls):
  edges = edge_index.astype(jnp.int32).reshape(2, EC, CHUNK)
  z64 = jnp.zeros((N_ACC, 64), jnp.float32)
  z32 = jnp.zeros((N_ACC, 32), jnp.float32)
  z16 = jnp.zeros((N_ACC, 16), jnp.float32)

  y1p = _tc0(x, W1l)
  r1 = _tcr(x, W1r, b1.reshape(1, 64))
  agg1, cnts = _sc_agg1(y1p, edges, z64)
  y2p, r2, cnt = _tc1(agg1, cnts, r1, W2l, W2r, b2.reshape(1, 32))
  agg2 = _sc_agg_32(y2p, edges, z32)
  y3p, r3 = _tc2(agg2, r2, cnt, W3l, W3r, b3.reshape(1, 16))
  agg3 = _sc_agg_16(y3p, edges, z16)
  wh = jnp.concatenate([Wreg, Wcls], axis=1)
  bh = jnp.stack([breg[0], bcls[0]]).reshape(1, 2)
  out = _tc3(agg3, r3, cnt, wh, bh)
  return out[:, 0], out[:, 1]


def kernel(x, edge_index, W1l, W1r, b1, W2l, W2r, b2, W3l, W3r, b3, Wreg,
           breg, Wcls, bcls):
  return _run(x, edge_index, W1l, W1r, b1, W2l, W2r, b2, W3l, W3r, b3, Wreg,
              breg, Wcls, bcls)


# speedup vs baseline: 2.6000x; 1.0464x over previous
import jax
import jax.numpy as jnp
from jax import lax
from jax.experimental import pallas as pl
from jax.experimental.pallas import tpu as pltpu
from jax.experimental.pallas import tpu_sc as plsc

N = 10000
E = 320000
NW = 32
CHUNK = 256
EC = E // CHUNK
WCH = EC // NW
C = WCH + 1
N_ACC = 10112
ROWS_PER_TILE = N_ACC // 16
N8 = N + 16
Y_ROWS_PER_TILE = N8 // 16


def _make_sc_agg(dw, stage_y):
  mesh = plsc.VectorSubcoreMesh(core_axis_name="c", subcore_axis_name="s")

  def body(y_hbm, edges_hbm, z_hbm, out_hbm, src_v, dst_v, rows_v,
           acc, y_spm, gsem):
    c = lax.axis_index("c")
    s = lax.axis_index("s")
    wid = s * 2 + c
    r0 = s * ROWS_PER_TILE
    stage = [(z_hbm.at[pl.ds(r0, ROWS_PER_TILE)],
              acc.at[pl.ds(r0, ROWS_PER_TILE)]),
             (edges_hbm.at[0, pl.ds(wid * WCH, WCH)],
              src_v.at[pl.ds(0, WCH)]),
             (edges_hbm.at[1, pl.ds(wid * WCH, WCH)],
              dst_v.at[pl.ds(0, WCH)])]
    if stage_y:
      y0 = s * Y_ROWS_PER_TILE
      stage.append((y_hbm.at[pl.ds(y0, Y_ROWS_PER_TILE)],
                    y_spm.at[pl.ds(y0, Y_ROWS_PER_TILE)]))
      y_src = y_spm
    else:
      y_src = y_hbm
    for src, dst in stage:
      pltpu.async_copy(src, dst, gsem)
    lax.cond(
        wid < EC - NW * WCH,
        lambda: (pltpu.sync_copy(edges_hbm.at[0, NW * WCH + wid],
                                 src_v.at[WCH]),
                 pltpu.sync_copy(edges_hbm.at[1, NW * WCH + wid],
                                 dst_v.at[WCH]))[0],
        lambda: None)
    nch = WCH + jnp.where(wid < EC - NW * WCH, 1, 0)
    for src, dst in stage:
      pltpu.make_async_copy(src, dst, gsem).wait()
    plsc.subcore_barrier()

    def step(j, carry):
      pltpu.async_copy(y_src.at[src_v.at[j]], rows_v, gsem).wait()
      pltpu.sync_copy(rows_v, acc.at[dst_v.at[j]], add=True)
      return carry

    lax.fori_loop(0, nch, step, 0)

    plsc.subcore_barrier()
    pltpu.sync_copy(acc.at[pl.ds(r0, ROWS_PER_TILE)],
                    out_hbm.at[c, pl.ds(r0, ROWS_PER_TILE)])

  scratch = [
      pltpu.VMEM((C, CHUNK), jnp.int32),
      pltpu.VMEM((C, CHUNK), jnp.int32),
      pltpu.VMEM((CHUNK, dw), jnp.float32),
      pltpu.VMEM_SHARED((N_ACC, dw), jnp.float32),
      pltpu.VMEM_SHARED((N8, dw) if stage_y else (8, dw), jnp.float32),
      pltpu.SemaphoreType.DMA,
  ]
  return pl.kernel(
      body,
      out_type=jax.ShapeDtypeStruct((2, N_ACC, dw), jnp.float32),
      mesh=mesh,
      compiler_params=pltpu.CompilerParams(use_tc_tiling_on_sc=False),
      scratch_types=scratch,
  )


_sc_agg_32 = _make_sc_agg(32, stage_y=True)
_sc_agg_16 = _make_sc_agg(16, stage_y=True)


def _make_sc_agg1():
  mesh = plsc.VectorSubcoreMesh(core_axis_name="c", subcore_axis_name="s")

  def body(y_hbm, edges_hbm, z_hbm, agg_out, cnt_out, src_v, dst_v,
           rows_v, cnt_v, acc, y_spm, gsem):
    c = lax.axis_index("c")
    s = lax.axis_index("s")
    wid = s * 2 + c
    r0 = s * ROWS_PER_TILE
    y0 = s * Y_ROWS_PER_TILE
    stage = [(z_hbm.at[pl.ds(r0, ROWS_PER_TILE)],
              acc.at[pl.ds(r0, ROWS_PER_TILE)]),
             (y_hbm.at[pl.ds(y0, Y_ROWS_PER_TILE)],
              y_spm.at[pl.ds(y0, Y_ROWS_PER_TILE)]),
             (edges_hbm.at[0, pl.ds(wid * WCH, WCH)],
              src_v.at[pl.ds(0, WCH)]),
             (edges_hbm.at[1, pl.ds(wid * WCH, WCH)],
              dst_v.at[pl.ds(0, WCH)])]
    for src, dst in stage:
      pltpu.async_copy(src, dst, gsem)
    lax.cond(
        wid < EC - NW * WCH,
        lambda: (pltpu.sync_copy(edges_hbm.at[0, NW * WCH + wid],
                                 src_v.at[WCH]),
                 pltpu.sync_copy(edges_hbm.at[1, NW * WCH + wid],
                                 dst_v.at[WCH]))[0],
        lambda: None)
    nch = WCH + jnp.where(wid < EC - NW * WCH, 1, 0)

    def zero_cnt(i, carry):
      cnt_v[pl.ds(16 * i, 16)] = jnp.zeros((16,), jnp.float32)
      return carry

    lax.fori_loop(0, N8 // 16, zero_cnt, 0)
    for src, dst in stage:
      pltpu.make_async_copy(src, dst, gsem).wait()
    plsc.subcore_barrier()

    ones16 = jnp.full((16,), 1.0, jnp.float32)

    def step(j, carry):
      cp = pltpu.async_copy(y_spm.at[src_v.at[j]], rows_v, gsem)
      drow = dst_v.at[j]
      for k in range(CHUNK // 16):
        idx = drow[pl.ds(16 * k, 16)]
        plsc.addupdate_scatter(cnt_v, [idx], ones16)
      cp.wait()
      pltpu.sync_copy(rows_v, acc.at[dst_v.at[j]], add=True)
      return carry

    lax.fori_loop(0, nch, step, 0)
    pltpu.sync_copy(cnt_v, cnt_out.at[wid])
    plsc.subcore_barrier()
    pltpu.sync_copy(acc.at[pl.ds(r0, ROWS_PER_TILE)],
                    agg_out.at[c, pl.ds(r0, ROWS_PER_TILE)])

  return pl.kernel(
      body,
      out_type=(jax.ShapeDtypeStruct((2, N_ACC, 64), jnp.float32),
                jax.ShapeDtypeStruct((NW, N8), jnp.float32)),
      mesh=mesh,
      compiler_params=pltpu.CompilerParams(use_tc_tiling_on_sc=False,
                                           needs_layout_passes=False),
      scratch_types=[
          pltpu.VMEM((C, CHUNK), jnp.int32),
          pltpu.VMEM((C, CHUNK), jnp.int32),
          pltpu.VMEM((CHUNK, 64), jnp.float32),
          pltpu.VMEM((N8,), jnp.float32),
          pltpu.VMEM_SHARED((N_ACC, 64), jnp.float32),
          pltpu.VMEM_SHARED((N8, 64), jnp.float32),
          pltpu.SemaphoreType.DMA,
      ],
  )


_sc_agg1 = _make_sc_agg1()


def _tc0_body(x_ref, w_ref, o_ref):
  y = jnp.dot(x_ref[...], w_ref[...], preferred_element_type=jnp.float32)
  o_ref[...] = jnp.concatenate([y, jnp.zeros((16, 64), jnp.float32)], axis=0)


_tc0 = pl.pallas_call(
    _tc0_body, out_shape=jax.ShapeDtypeStruct((N8, 64), jnp.float32))


def _tcr_body(h_ref, w_ref, b_ref, r_ref):
  r_ref[...] = jnp.dot(h_ref[...], w_ref[...],
                       preferred_element_type=jnp.float32) + b_ref[...]


_tcr = pl.pallas_call(
    _tcr_body, out_shape=jax.ShapeDtypeStruct((N, 64), jnp.float32))


def _tc1_body(agg_ref, cnts_ref, r1_ref, wl2_ref, wr2_ref, b2_ref,
              y2_ref, r2_ref, cnt_ref):
  a = agg_ref[0, :N] + agg_ref[1, :N]
  ccol = lax.dot_general(cnts_ref[...], jnp.ones((NW, 1), jnp.float32),
                         dimension_numbers=(((0,), (0,)), ((), ())))
  cnt = jnp.maximum(ccol[:N], 1.0)
  h = jnp.maximum(a / cnt + r1_ref[...], 0.0)
  y2 = jnp.dot(h, wl2_ref[...], preferred_element_type=jnp.float32)
  y2_ref[...] = jnp.concatenate([y2, jnp.zeros((16, 32), jnp.float32)], axis=0)
  r2_ref[...] = jnp.dot(h, wr2_ref[...],
                        preferred_element_type=jnp.float32) + b2_ref[...]
  cnt_ref[...] = cnt


_tc1 = pl.pallas_call(
    _tc1_body,
    out_shape=(
        jax.ShapeDtypeStruct((N8, 32), jnp.float32),
        jax.ShapeDtypeStruct((N, 32), jnp.float32),
        jax.ShapeDtypeStruct((N, 1), jnp.float32),
    ))


def _tc2_body(agg_ref, r2_ref, cnt_ref, wl3_ref, wr3_ref, b3_ref,
              y3_ref, r3_ref):
  a = agg_ref[0, :N] + agg_ref[1, :N]
  h = jnp.maximum(a / cnt_ref[...] + r2_ref[...], 0.0)
  y3 = jnp.dot(h, wl3_ref[...], preferred_element_type=jnp.float32)
  y3_ref[...] = jnp.concatenate([y3, jnp.zeros((16, 16), jnp.float32)], axis=0)
  r3_ref[...] = jnp.dot(h, wr3_ref[...],
                        preferred_element_type=jnp.float32) + b3_ref[...]


_tc2 = pl.pallas_call(
    _tc2_body,
    out_shape=(
        jax.ShapeDtypeStruct((N8, 16), jnp.float32),
        jax.ShapeDtypeStruct((N, 16), jnp.float32),
    ))


def _tc3_body(agg_ref, r3_ref, cnt_ref, wh_ref, bh_ref, o_ref):
  a = agg_ref[0, :N] + agg_ref[1, :N]
  h = jnp.maximum(a / cnt_ref[...] + r3_ref[...], 0.0)
  o_ref[...] = jnp.dot(
      h, wh_ref[...], preferred_element_type=jnp.float32) + bh_ref[...]


_tc3 = pl.pallas_call(
    _tc3_body, out_shape=jax.ShapeDtypeStruct((N, 2), jnp.float32))


@jax.jit
def _run(x, edge_index, W1l, W1r, b1, W2l, W2r, b2, W3l, W3r, b3, Wreg, breg,
         Wcls, bcls):
  edges = edge_index.astype(jnp.int32).reshape(2, EC, CHUNK)
  z64 = jnp.zeros((N_ACC, 64), jnp.float32)
  z32 = jnp.zeros((N_ACC, 32), jnp.float32)
  z16 = jnp.zeros((N_ACC, 16), jnp.float32)

  y1p = _tc0(x, W1l)
  r1 = _tcr(x, W1r, b1.reshape(1, 64))
  agg1, cnts = _sc_agg1(y1p, edges, z64)
  y2p, r2, cnt = _tc1(agg1, cnts, r1, W2l, W2r, b2.reshape(1, 32))
  agg2 = _sc_agg_32(y2p, edges, z32)
  y3p, r3 = _tc2(agg2, r2, cnt, W3l, W3r, b3.reshape(1, 16))
  agg3 = _sc_agg_16(y3p, edges, z16)
  wh = jnp.concatenate([Wreg, Wcls], axis=1)
  bh = jnp.stack([breg[0], bcls[0]]).reshape(1, 2)
  out = _tc3(agg3, r3, cnt, wh, bh)
  return out[:, 0], out[:, 1]


def kernel(x, edge_index, W1l, W1r, b1, W2l, W2r, b2, W3l, W3r, b3, Wreg,
           breg, Wcls, bcls):
  return _run(x, edge_index, W1l, W1r, b1, W2l, W2r, b2, W3l, W3r, b3, Wreg,
              breg, Wcls, bcls)


# speedup vs baseline: 2.6704x; 1.0271x over previous
import jax
import jax.numpy as jnp
from jax import lax
from jax.experimental import pallas as pl
from jax.experimental.pallas import tpu as pltpu
from jax.experimental.pallas import tpu_sc as plsc

N = 10000
E = 320000
NW = 32
CHUNK = 256
EC = E // CHUNK
WCH = EC // NW
C = WCH + 1
N_ACC = 10112
ROWS_PER_TILE = N_ACC // 16
N8 = N + 16
Y_ROWS_PER_TILE = N8 // 16


def _make_sc_agg(dw, stage_y):
  mesh = plsc.VectorSubcoreMesh(core_axis_name="c", subcore_axis_name="s")

  def body(y_hbm, edges_hbm, z_hbm, out_hbm, src_v, dst_v, rows_v,
           acc, y_spm, gsem):
    c = lax.axis_index("c")
    s = lax.axis_index("s")
    wid = s * 2 + c
    r0 = s * ROWS_PER_TILE
    stage = [(z_hbm.at[pl.ds(r0, ROWS_PER_TILE)],
              acc.at[pl.ds(r0, ROWS_PER_TILE)]),
             (edges_hbm.at[0, pl.ds(wid * WCH, WCH)],
              src_v.at[pl.ds(0, WCH)]),
             (edges_hbm.at[1, pl.ds(wid * WCH, WCH)],
              dst_v.at[pl.ds(0, WCH)])]
    if stage_y:
      y0 = s * Y_ROWS_PER_TILE
      stage.append((y_hbm.at[pl.ds(y0, Y_ROWS_PER_TILE)],
                    y_spm.at[pl.ds(y0, Y_ROWS_PER_TILE)]))
      y_src = y_spm
    else:
      y_src = y_hbm
    for src, dst in stage:
      pltpu.async_copy(src, dst, gsem)
    lax.cond(
        wid < EC - NW * WCH,
        lambda: (pltpu.sync_copy(edges_hbm.at[0, NW * WCH + wid],
                                 src_v.at[WCH]),
                 pltpu.sync_copy(edges_hbm.at[1, NW * WCH + wid],
                                 dst_v.at[WCH]))[0],
        lambda: None)
    nch = WCH + jnp.where(wid < EC - NW * WCH, 1, 0)
    for src, dst in stage:
      pltpu.make_async_copy(src, dst, gsem).wait()
    plsc.subcore_barrier()

    def step(j, carry):
      pltpu.async_copy(y_src.at[src_v.at[j]], rows_v, gsem).wait()
      pltpu.sync_copy(rows_v, acc.at[dst_v.at[j]], add=True)
      return carry

    lax.fori_loop(0, nch, step, 0)

    plsc.subcore_barrier()
    pltpu.sync_copy(acc.at[pl.ds(r0, ROWS_PER_TILE)],
                    out_hbm.at[c, pl.ds(r0, ROWS_PER_TILE)])

  scratch = [
      pltpu.VMEM((C, CHUNK), jnp.int32),
      pltpu.VMEM((C, CHUNK), jnp.int32),
      pltpu.VMEM((CHUNK, dw), jnp.float32),
      pltpu.VMEM_SHARED((N_ACC, dw), jnp.float32),
      pltpu.VMEM_SHARED((N8, dw) if stage_y else (8, dw), jnp.float32),
      pltpu.SemaphoreType.DMA,
  ]
  return pl.kernel(
      body,
      out_type=jax.ShapeDtypeStruct((2, N_ACC, dw), jnp.float32),
      mesh=mesh,
      compiler_params=pltpu.CompilerParams(use_tc_tiling_on_sc=False),
      scratch_types=scratch,
  )


_sc_agg_32 = _make_sc_agg(32, stage_y=True)
_sc_agg_16 = _make_sc_agg(16, stage_y=True)


def _make_sc_agg1():
  mesh = plsc.VectorSubcoreMesh(core_axis_name="c", subcore_axis_name="s")

  def body(y_hbm, edges_hbm, z_hbm, agg_out, cnt_out, src_v, dst_v,
           rows_v, cnt_v, acc, y_spm, gsem):
    c = lax.axis_index("c")
    s = lax.axis_index("s")
    wid = s * 2 + c
    r0 = s * ROWS_PER_TILE
    y0 = s * Y_ROWS_PER_TILE
    stage = [(z_hbm.at[pl.ds(r0, ROWS_PER_TILE)],
              acc.at[pl.ds(r0, ROWS_PER_TILE)]),
             (y_hbm.at[pl.ds(y0, Y_ROWS_PER_TILE)],
              y_spm.at[pl.ds(y0, Y_ROWS_PER_TILE)]),
             (edges_hbm.at[0, pl.ds(wid * WCH, WCH)],
              src_v.at[pl.ds(0, WCH)]),
             (edges_hbm.at[1, pl.ds(wid * WCH, WCH)],
              dst_v.at[pl.ds(0, WCH)])]
    for src, dst in stage:
      pltpu.async_copy(src, dst, gsem)
    lax.cond(
        wid < EC - NW * WCH,
        lambda: (pltpu.sync_copy(edges_hbm.at[0, NW * WCH + wid],
                                 src_v.at[WCH]),
                 pltpu.sync_copy(edges_hbm.at[1, NW * WCH + wid],
                                 dst_v.at[WCH]))[0],
        lambda: None)
    nch = WCH + jnp.where(wid < EC - NW * WCH, 1, 0)

    def zero_cnt(i, carry):
      cnt_v[pl.ds(16 * i, 16)] = jnp.zeros((16,), jnp.float32)
      return carry

    lax.fori_loop(0, N8 // 16, zero_cnt, 0)
    for src, dst in stage:
      pltpu.make_async_copy(src, dst, gsem).wait()
    plsc.subcore_barrier()

    ones16 = jnp.full((16,), 1.0, jnp.float32)

    def step(j, carry):
      cp = pltpu.async_copy(y_spm.at[src_v.at[j]], rows_v, gsem)
      drow = dst_v.at[j]
      for k in range(CHUNK // 16):
        idx = drow[pl.ds(16 * k, 16)]
        plsc.addupdate_scatter(cnt_v, [idx], ones16)
      cp.wait()
      pltpu.sync_copy(rows_v, acc.at[dst_v.at[j]], add=True)
      return carry

    lax.fori_loop(0, nch, step, 0)
    pltpu.sync_copy(cnt_v, cnt_out.at[wid])
    plsc.subcore_barrier()
    pltpu.sync_copy(acc.at[pl.ds(r0, ROWS_PER_TILE)],
                    agg_out.at[c, pl.ds(r0, ROWS_PER_TILE)])

  return pl.kernel(
      body,
      out_type=(jax.ShapeDtypeStruct((2, N_ACC, 64), jnp.float32),
                jax.ShapeDtypeStruct((NW, N8), jnp.float32)),
      mesh=mesh,
      compiler_params=pltpu.CompilerParams(use_tc_tiling_on_sc=False,
                                           needs_layout_passes=False),
      scratch_types=[
          pltpu.VMEM((C, CHUNK), jnp.int32),
          pltpu.VMEM((C, CHUNK), jnp.int32),
          pltpu.VMEM((CHUNK, 64), jnp.float32),
          pltpu.VMEM((N8,), jnp.float32),
          pltpu.VMEM_SHARED((N_ACC, 64), jnp.float32),
          pltpu.VMEM_SHARED((N8, 64), jnp.float32),
          pltpu.SemaphoreType.DMA,
      ],
  )


_sc_agg1 = _make_sc_agg1()


def _tc0_body(x_ref, w_ref, o_ref):
  y = jnp.dot(x_ref[...], w_ref[...], preferred_element_type=jnp.float32)
  o_ref[...] = jnp.concatenate([y, jnp.zeros((16, 64), jnp.float32)], axis=0)


_tc0 = pl.pallas_call(
    _tc0_body, out_shape=jax.ShapeDtypeStruct((N8, 64), jnp.float32))


def _tcr_body(h_ref, w_ref, b_ref, r_ref):
  r_ref[...] = jnp.dot(h_ref[...], w_ref[...],
                       preferred_element_type=jnp.float32) + b_ref[...]


_tcr = pl.pallas_call(
    _tcr_body, out_shape=jax.ShapeDtypeStruct((N, 64), jnp.float32))


def _tc1_body(agg_ref, cnts_ref, r1_ref, wl2_ref, wr2_ref, b2_ref,
              y2_ref, r2_ref, cnt_ref):
  a = agg_ref[0, :N] + agg_ref[1, :N]
  ccol = lax.dot_general(cnts_ref[...], jnp.ones((NW, 1), jnp.float32),
                         dimension_numbers=(((0,), (0,)), ((), ())))
  cnt = jnp.maximum(ccol[:N], 1.0)
  h = jnp.maximum(a / cnt + r1_ref[...], 0.0)
  y2 = jnp.dot(h, wl2_ref[...], preferred_element_type=jnp.float32)
  y2_ref[...] = jnp.concatenate([y2, jnp.zeros((16, 32), jnp.float32)], axis=0)
  r2_ref[...] = jnp.dot(h, wr2_ref[...],
                        preferred_element_type=jnp.float32) + b2_ref[...]
  cnt_ref[...] = cnt


_tc1 = pl.pallas_call(
    _tc1_body,
    out_shape=(
        jax.ShapeDtypeStruct((N8, 32), jnp.float32),
        jax.ShapeDtypeStruct((N, 32), jnp.float32),
        jax.ShapeDtypeStruct((N, 1), jnp.float32),
    ))


def _tc2_body(agg_ref, r2_ref, cnt_ref, wl3_ref, wr3_ref, b3_ref,
              y3_ref, r3_ref):
  a = agg_ref[0, :N] + agg_ref[1, :N]
  h = jnp.maximum(a / cnt_ref[...] + r2_ref[...], 0.0)
  y3 = jnp.dot(h, wl3_ref[...], preferred_element_type=jnp.float32)
  y3_ref[...] = jnp.concatenate([y3, jnp.zeros((16, 16), jnp.float32)], axis=0)
  r3_ref[...] = jnp.dot(h, wr3_ref[...],
                        preferred_element_type=jnp.float32) + b3_ref[...]


_tc2 = pl.pallas_call(
    _tc2_body,
    out_shape=(
        jax.ShapeDtypeStruct((N8, 16), jnp.float32),
        jax.ShapeDtypeStruct((N, 16), jnp.float32),
    ))


def _tc3_body(agg_ref, r3_ref, cnt_ref, wh_ref, bh_ref, o_ref):
  a = agg_ref[0, :N] + agg_ref[1, :N]
  h = jnp.maximum(a / cnt_ref[...] + r3_ref[...], 0.0)
  o_ref[...] = lax.dot_general(
      wh_ref[...], h, dimension_numbers=(((0,), (1,)), ((), ())),
      preferred_element_type=jnp.float32) + bh_ref[...]


_tc3 = pl.pallas_call(
    _tc3_body, out_shape=jax.ShapeDtypeStruct((2, N), jnp.float32))


@jax.jit
def _run(x, edge_index, W1l, W1r, b1, W2l, W2r, b2, W3l, W3r, b3, Wreg, breg,
         Wcls, bcls):
  edges = edge_index.astype(jnp.int32).reshape(2, EC, CHUNK)
  z64 = jnp.zeros((N_ACC, 64), jnp.float32)
  z32 = jnp.zeros((N_ACC, 32), jnp.float32)
  z16 = jnp.zeros((N_ACC, 16), jnp.float32)

  y1p = _tc0(x, W1l)
  r1 = _tcr(x, W1r, b1.reshape(1, 64))
  agg1, cnts = _sc_agg1(y1p, edges, z64)
  y2p, r2, cnt = _tc1(agg1, cnts, r1, W2l, W2r, b2.reshape(1, 32))
  agg2 = _sc_agg_32(y2p, edges, z32)
  y3p, r3 = _tc2(agg2, r2, cnt, W3l, W3r, b3.reshape(1, 16))
  agg3 = _sc_agg_16(y3p, edges, z16)
  wh = jnp.concatenate([Wreg, Wcls], axis=1)
  bh = jnp.stack([breg[0], bcls[0]]).reshape(2, 1)
  out = _tc3(agg3, r3, cnt, wh, bh)
  return out[0], out[1]


def kernel(x, edge_index, W1l, W1r, b1, W2l, W2r, b2, W3l, W3r, b3, Wreg,
           breg, Wcls, bcls):
  return _run(x, edge_index, W1l, W1r, b1, W2l, W2r, b2, W3l, W3r, b3, Wreg,
              breg, Wcls, bcls)
